# baseline (speedup 1.0000x reference)
import functools

import jax
import jax.numpy as jnp
from jax import lax
from jax.experimental import pallas as pl
from jax.experimental.pallas import tpu as pltpu
from jax.experimental.pallas import tpu_sc as plsc

N_NODES = 10000
N_EDGES = 320000
D_IN = 128

NC = 2
NS = 16
CW = 64
K = 128
NB = 2
EPW = N_EDGES // NS
NCHUNK = -(-EPW // K)
EPW_PAD = NCHUNK * K
N_PAD = N_NODES + 16
RPT = N_NODES // NS


def _make_sc_agg(with_count):
    mesh = plsc.VectorSubcoreMesh(
        core_axis_name="c", subcore_axis_name="s", num_cores=NC, num_subcores=NS
    )
    out_type = [jax.ShapeDtypeStruct((N_NODES, NC * CW), jnp.float32)]
    scratch = [
        pltpu.VMEM((NCHUNK, K), jnp.int32),
        pltpu.VMEM((NCHUNK, K), jnp.int32),
        [pltpu.VMEM((K, CW), jnp.float32) for _ in range(NB)],
        pltpu.VMEM_SHARED((N_PAD, CW), jnp.float32),
        [pltpu.SemaphoreType.DMA for _ in range(NB)],
    ]
    if with_count:
        out_type.append(jax.ShapeDtypeStruct((N_NODES,), jnp.float32))
        scratch.append(pltpu.VMEM((K,), jnp.float32))
        scratch.append(pltpu.VMEM_SHARED((N_PAD,), jnp.float32))

    @functools.partial(
        pl.kernel,
        out_type=out_type,
        mesh=mesh,
        compiler_params=pltpu.CompilerParams(use_tc_tiling_on_sc=False),
        scratch_types=scratch,
    )
    def agg(tbl_hbm, src_hbm, dst_hbm, zeros_hbm, *rest):
        if with_count:
            (zcnt_hbm, out_hbm, cnt_hbm,
             sidx, didx, rows, acc, gsem, ones, cnt) = rest
        else:
            (out_hbm, sidx, didx, rows, acc, gsem) = rest
        c = lax.axis_index("c")
        s = lax.axis_index("s")
        r0 = s * RPT
        pltpu.sync_copy(src_hbm.at[s], sidx)
        pltpu.sync_copy(dst_hbm.at[s], didx)
        pltpu.sync_copy(zeros_hbm.at[pl.ds(r0, RPT)], acc.at[pl.ds(r0, RPT)])
        if with_count:
            @pl.when(s == 0)
            def _():
                pltpu.sync_copy(zcnt_hbm, cnt.at[pl.ds(0, N_NODES)])
            for v in range(K // 16):
                ones[pl.ds(v * 16, 16)] = jnp.ones((16,), jnp.float32)

        def tbody(r, carry):
            for u in range(K // 16):
                sl = pl.ds(u * 16, 16)
                sidx[r, sl] = sidx[r, sl] * 2 + c
            return carry

        lax.fori_loop(0, NCHUNK, tbody, 0)
        plsc.subcore_barrier()

        def gfire(i, b):
            pltpu.async_copy(tbl_hbm.at[sidx.at[i]], rows[b], gsem[b])

        def gwait(i, b):
            pltpu.make_async_copy(tbl_hbm.at[sidx.at[i]], rows[b], gsem[b]).wait()

        def scatter(i, b):
            pltpu.sync_copy(rows[b], acc.at[didx.at[i]], add=True)
            if with_count:
                pltpu.sync_copy(ones, cnt.at[didx.at[i]], add=True)

        gfire(0, 0)

        def body(j, carry):
            a = 2 * j
            gfire(a + 1, 1)
            gwait(a, 0)
            scatter(a, 0)
            gfire(a + 2, 0)
            gwait(a + 1, 1)
            scatter(a + 1, 1)
            return carry

        lax.fori_loop(0, (NCHUNK - 1) // 2, body, 0)
        gwait(NCHUNK - 1, 0)
        scatter(NCHUNK - 1, 0)

        plsc.subcore_barrier()
        pltpu.sync_copy(acc.at[pl.ds(r0, RPT)],
                        out_hbm.at[pl.ds(r0, RPT), pl.ds(c * CW, CW)])
        if with_count:
            @pl.when((c == 0) & (s == 0))
            def _():
                pltpu.sync_copy(cnt.at[pl.ds(0, N_NODES)], cnt_hbm)

    return agg


_agg_l1 = _make_sc_agg(True)
_agg_l2 = _make_sc_agg(False)

BLK = 1000


def _dense1_body(a_ref, cnt_ref, x_ref, wl, b, wr, h_ref, inv_ref):
    inv = 1.0 / jnp.maximum(cnt_ref[...], 1.0)
    mean = a_ref[...] * inv
    h = (jnp.dot(mean, wl[...], preferred_element_type=jnp.float32)
         + b[...]
         + jnp.dot(x_ref[...], wr[...], preferred_element_type=jnp.float32))
    h_ref[...] = jnp.maximum(h, 0.0)
    inv_ref[...] = inv


def _dense1(acc, cnt2d, x, wlT, b2d, wrT):
    grid = (N_NODES // BLK,)
    return pl.pallas_call(
        _dense1_body,
        grid=grid,
        in_specs=[
            pl.BlockSpec((BLK, D_IN), lambda i: (i, 0)),
            pl.BlockSpec((BLK, 1), lambda i: (i, 0)),
            pl.BlockSpec((BLK, D_IN), lambda i: (i, 0)),
            pl.BlockSpec((D_IN, D_IN), lambda i: (0, 0)),
            pl.BlockSpec((1, D_IN), lambda i: (0, 0)),
            pl.BlockSpec((D_IN, D_IN), lambda i: (0, 0)),
        ],
        out_specs=[
            pl.BlockSpec((BLK, D_IN), lambda i: (i, 0)),
            pl.BlockSpec((BLK, 1), lambda i: (i, 0)),
        ],
        out_shape=[
            jax.ShapeDtypeStruct((N_NODES, D_IN), jnp.float32),
            jax.ShapeDtypeStruct((N_NODES, 1), jnp.float32),
        ],
    )(acc, cnt2d, x, wlT, b2d, wrT)


def _dense2_body(a_ref, inv_ref, h_ref, wl, b, wr, o_ref):
    mean = a_ref[...] * inv_ref[...]
    o_ref[...] = (jnp.dot(mean, wl[...], preferred_element_type=jnp.float32)
                  + b[...]
                  + jnp.dot(h_ref[...], wr[...], preferred_element_type=jnp.float32))


def _dense2(acc, inv, h, wlT, b2d, wrT):
    grid = (N_NODES // BLK,)
    return pl.pallas_call(
        _dense2_body,
        grid=grid,
        in_specs=[
            pl.BlockSpec((BLK, D_IN), lambda i: (i, 0)),
            pl.BlockSpec((BLK, 1), lambda i: (i, 0)),
            pl.BlockSpec((BLK, D_IN), lambda i: (i, 0)),
            pl.BlockSpec((D_IN, D_IN), lambda i: (0, 0)),
            pl.BlockSpec((1, D_IN), lambda i: (0, 0)),
            pl.BlockSpec((D_IN, D_IN), lambda i: (0, 0)),
        ],
        out_specs=pl.BlockSpec((BLK, D_IN), lambda i: (i, 0)),
        out_shape=jax.ShapeDtypeStruct((N_NODES, D_IN), jnp.float32),
    )(acc, inv, h, wlT, b2d, wrT)


def _pad_idx(v, trash):
    v = v.reshape(NS, EPW)
    npad = EPW_PAD - EPW
    if trash:
        fill = N_NODES + (jnp.arange(npad, dtype=jnp.int32) % 16)
    else:
        fill = jnp.zeros((npad,), jnp.int32)
    pad = jnp.broadcast_to(fill, (NS, npad))
    return jnp.concatenate([v, pad], axis=1).reshape(NS, NCHUNK, K)


def kernel(x, edge_index, Wl1, bl1, Wr1, Wl2, bl2, Wr2):
    src = _pad_idx(edge_index[0].astype(jnp.int32), False)
    dst = _pad_idx(edge_index[1].astype(jnp.int32), True)
    z64 = jnp.zeros((N_NODES, CW), jnp.float32)
    zcnt = jnp.zeros((N_NODES,), jnp.float32)

    x2 = x.reshape(2 * N_NODES, CW)
    acc1, cnt = _agg_l1(x2, src, dst, z64, zcnt)
    h, inv = _dense1(acc1, cnt[:, None], x, Wl1.T, bl1[None, :], Wr1.T)
    acc2, = _agg_l2(h.reshape(2 * N_NODES, CW), src, dst, z64)
    out = _dense2(acc2, inv, h, Wl2.T, bl2[None, :], Wr2.T)
    return out


# speedup vs baseline: 1.0701x; 1.0701x over previous
import functools

import jax
import jax.numpy as jnp
from jax import lax
from jax.experimental import pallas as pl
from jax.experimental.pallas import tpu as pltpu
from jax.experimental.pallas import tpu_sc as plsc

N_NODES = 10000
N_EDGES = 320000
D_IN = 128

NC = 2
NS = 16
CW = 64
K = 128
EPW = N_EDGES // NS
NCHUNK = -(-EPW // K)
EPW_PAD = NCHUNK * K
N_PAD = N_NODES + 16
RPT = N_NODES // NS


def _make_sc_agg(with_count):
    mesh = plsc.VectorSubcoreMesh(
        core_axis_name="c", subcore_axis_name="s", num_cores=NC, num_subcores=NS
    )
    out_type = [jax.ShapeDtypeStruct((N_NODES, NC * CW), jnp.float32)]
    scratch = [
        pltpu.VMEM((NCHUNK, K), jnp.int32),
        pltpu.VMEM((NCHUNK, K), jnp.int32),
        pltpu.VMEM((K, CW), jnp.float32),
        pltpu.VMEM((K, CW), jnp.float32),
        pltpu.VMEM_SHARED((N_PAD, CW), jnp.float32),
        pltpu.SemaphoreType.DMA,
        pltpu.SemaphoreType.DMA,
    ]
    if with_count:
        out_type.append(jax.ShapeDtypeStruct((N_NODES,), jnp.float32))
        scratch.append(pltpu.VMEM((K,), jnp.float32))
        scratch.append(pltpu.VMEM_SHARED((N_PAD,), jnp.float32))

    @functools.partial(
        pl.kernel,
        out_type=out_type,
        mesh=mesh,
        compiler_params=pltpu.CompilerParams(use_tc_tiling_on_sc=False),
        scratch_types=scratch,
    )
    def agg(tbl_hbm, src_hbm, dst_hbm, zeros_hbm, *rest):
        if with_count:
            (zcnt_hbm, out_hbm, cnt_hbm,
             sidx, didx, rows0, rows1, acc, sem0, sem1, ones, cnt) = rest
        else:
            (out_hbm, sidx, didx, rows0, rows1, acc, sem0, sem1) = rest
        c = lax.axis_index("c")
        s = lax.axis_index("s")
        r0 = s * RPT
        pltpu.sync_copy(src_hbm.at[s], sidx)
        pltpu.sync_copy(dst_hbm.at[s], didx)
        pltpu.sync_copy(zeros_hbm.at[pl.ds(r0, RPT)], acc.at[pl.ds(r0, RPT)])
        if with_count:
            @pl.when(s == 0)
            def _():
                pltpu.sync_copy(zcnt_hbm, cnt.at[pl.ds(0, N_NODES)])
            for v in range(K // 16):
                ones[pl.ds(v * 16, 16)] = jnp.ones((16,), jnp.float32)
        plsc.subcore_barrier()

        def gather(i, rows, sem):
            pltpu.async_copy(tbl_hbm.at[c].at[sidx.at[i]], rows, sem)

        def gwait(i, rows, sem):
            pltpu.make_async_copy(tbl_hbm.at[c].at[sidx.at[i]], rows, sem).wait()

        def scatter(i, rows):
            pltpu.sync_copy(rows, acc.at[didx.at[i]], add=True)
            if with_count:
                pltpu.sync_copy(ones, cnt.at[didx.at[i]], add=True)

        gather(0, rows0, sem0)

        def body(j, carry):
            a = 2 * j
            gather(a + 1, rows1, sem1)
            gwait(a, rows0, sem0)
            scatter(a, rows0)
            gather(a + 2, rows0, sem0)
            gwait(a + 1, rows1, sem1)
            scatter(a + 1, rows1)
            return carry

        lax.fori_loop(0, (NCHUNK - 1) // 2, body, 0)
        gwait(NCHUNK - 1, rows0, sem0)
        scatter(NCHUNK - 1, rows0)

        plsc.subcore_barrier()
        pltpu.sync_copy(acc.at[pl.ds(r0, RPT)],
                        out_hbm.at[pl.ds(r0, RPT), pl.ds(c * CW, CW)])
        if with_count:
            @pl.when((c == 0) & (s == 0))
            def _():
                pltpu.sync_copy(cnt.at[pl.ds(0, N_NODES)], cnt_hbm)

    return agg


_agg_l1 = _make_sc_agg(True)
_agg_l2 = _make_sc_agg(False)

BLK = 1000

_WSPECS = [
    pl.BlockSpec((D_IN, D_IN), lambda i: (0, 0)),
    pl.BlockSpec((1, D_IN), lambda i: (0, 0)),
]


def _xr1_body(x_ref, wr, b, o_ref):
    o_ref[...] = (jnp.dot(x_ref[...], wr[...], preferred_element_type=jnp.float32)
                  + b[...])


def _xr1(x, wrT, b2d):
    return pl.pallas_call(
        _xr1_body,
        grid=(N_NODES // BLK,),
        in_specs=[pl.BlockSpec((BLK, D_IN), lambda i: (i, 0))] + _WSPECS,
        out_specs=pl.BlockSpec((BLK, D_IN), lambda i: (i, 0)),
        out_shape=jax.ShapeDtypeStruct((N_NODES, D_IN), jnp.float32),
    )(x, wrT, b2d)


def _xr2_body(h_ref, wr, b, o_ref):
    h = jnp.concatenate([h_ref[0], h_ref[1]], axis=1)
    o_ref[...] = (jnp.dot(h, wr[...], preferred_element_type=jnp.float32)
                  + b[...])


def _xr2(h_tbl, wrT, b2d):
    return pl.pallas_call(
        _xr2_body,
        grid=(N_NODES // BLK,),
        in_specs=[pl.BlockSpec((2, BLK, CW), lambda i: (0, i, 0))] + _WSPECS,
        out_specs=pl.BlockSpec((BLK, D_IN), lambda i: (i, 0)),
        out_shape=jax.ShapeDtypeStruct((N_NODES, D_IN), jnp.float32),
    )(h_tbl, wrT, b2d)


def _dense1_body(a_ref, cnt_ref, xr_ref, wl, h_ref, inv_ref):
    inv = 1.0 / jnp.maximum(cnt_ref[...], 1.0)
    mean = a_ref[...] * inv
    h = (jnp.dot(mean, wl[...], preferred_element_type=jnp.float32)
         + xr_ref[...])
    h = jnp.maximum(h, 0.0)
    h_ref[0] = h[:, :CW]
    h_ref[1] = h[:, CW:]
    inv_ref[...] = inv


def _dense1(acc, cnt2d, xr, wlT):
    return pl.pallas_call(
        _dense1_body,
        grid=(N_NODES // BLK,),
        in_specs=[
            pl.BlockSpec((BLK, D_IN), lambda i: (i, 0)),
            pl.BlockSpec((BLK, 1), lambda i: (i, 0)),
            pl.BlockSpec((BLK, D_IN), lambda i: (i, 0)),
            pl.BlockSpec((D_IN, D_IN), lambda i: (0, 0)),
        ],
        out_specs=[
            pl.BlockSpec((2, BLK, CW), lambda i: (0, i, 0)),
            pl.BlockSpec((BLK, 1), lambda i: (i, 0)),
        ],
        out_shape=[
            jax.ShapeDtypeStruct((2, N_NODES, CW), jnp.float32),
            jax.ShapeDtypeStruct((N_NODES, 1), jnp.float32),
        ],
    )(acc, cnt2d, xr, wlT)


def _dense2_body(a_ref, inv_ref, hr_ref, wl, o_ref):
    mean = a_ref[...] * inv_ref[...]
    o_ref[...] = (jnp.dot(mean, wl[...], preferred_element_type=jnp.float32)
                  + hr_ref[...])


def _dense2(acc, inv, hr, wlT):
    return pl.pallas_call(
        _dense2_body,
        grid=(N_NODES // BLK,),
        in_specs=[
            pl.BlockSpec((BLK, D_IN), lambda i: (i, 0)),
            pl.BlockSpec((BLK, 1), lambda i: (i, 0)),
            pl.BlockSpec((BLK, D_IN), lambda i: (i, 0)),
            pl.BlockSpec((D_IN, D_IN), lambda i: (0, 0)),
        ],
        out_specs=pl.BlockSpec((BLK, D_IN), lambda i: (i, 0)),
        out_shape=jax.ShapeDtypeStruct((N_NODES, D_IN), jnp.float32),
    )(acc, inv, hr, wlT)


def _pad_idx(v, trash):
    v = v.reshape(NS, EPW)
    npad = EPW_PAD - EPW
    if trash:
        fill = N_NODES + (jnp.arange(npad, dtype=jnp.int32) % 16)
    else:
        fill = jnp.zeros((npad,), jnp.int32)
    pad = jnp.broadcast_to(fill, (NS, npad))
    return jnp.concatenate([v, pad], axis=1).reshape(NS, NCHUNK, K)


def kernel(x, edge_index, Wl1, bl1, Wr1, Wl2, bl2, Wr2):
    src = _pad_idx(edge_index[0].astype(jnp.int32), False)
    dst = _pad_idx(edge_index[1].astype(jnp.int32), True)
    tbl1 = jnp.stack([x[:, :CW], x[:, CW:]])
    z64 = jnp.zeros((N_NODES, CW), jnp.float32)
    zcnt = jnp.zeros((N_NODES,), jnp.float32)

    acc1, cnt = _agg_l1(tbl1, src, dst, z64, zcnt)
    xr1 = _xr1(x, Wr1.T, bl1[None, :])
    h_tbl, inv = _dense1(acc1, cnt[:, None], xr1, Wl1.T)
    acc2, = _agg_l2(h_tbl, src, dst, z64)
    hr2 = _xr2(h_tbl, Wr2.T, bl2[None, :])
    out = _dense2(acc2, inv, hr2, Wl2.T)
    return out


# speedup vs baseline: 1.2662x; 1.1833x over previous
import functools

import jax
import jax.numpy as jnp
from jax import lax
from jax.experimental import pallas as pl
from jax.experimental.pallas import tpu as pltpu
from jax.experimental.pallas import tpu_sc as plsc

N_NODES = 10000
N_EDGES = 320000
D_IN = 128

NC = 2
NS = 16
CW = 64
K = 128
EPW = N_EDGES // NS
NCHUNK = -(-EPW // K)
EPW_PAD = NCHUNK * K
N_PAD = N_NODES + 16
RPT = N_NODES // NS


def _make_sc_agg(with_count):
    mesh = plsc.VectorSubcoreMesh(
        core_axis_name="c", subcore_axis_name="s", num_cores=NC, num_subcores=NS
    )
    out_type = [jax.ShapeDtypeStruct((N_NODES, NC * CW), jnp.float32)]
    scratch = [
        pltpu.VMEM((NCHUNK, K), jnp.int32),
        pltpu.VMEM((NCHUNK, K), jnp.int32),
        [pltpu.VMEM((K, CW), jnp.float32) for _ in range(4)],
        pltpu.VMEM_SHARED((N_PAD, CW), jnp.float32),
        [pltpu.SemaphoreType.DMA for _ in range(4)],
    ]
    if with_count:
        out_type.append(jax.ShapeDtypeStruct((N_NODES,), jnp.float32))
        scratch.append(pltpu.VMEM((K,), jnp.float32))
        scratch.append(pltpu.VMEM_SHARED((N_PAD,), jnp.float32))

    @functools.partial(
        pl.kernel,
        out_type=out_type,
        mesh=mesh,
        compiler_params=pltpu.CompilerParams(use_tc_tiling_on_sc=False),
        scratch_types=scratch,
    )
    def agg(tbl_hbm, src_hbm, dst_hbm, zeros_hbm, *rest):
        if with_count:
            (zcnt_hbm, out_hbm, cnt_hbm,
             sidx, didx, rows, acc, gsem, ones, cnt) = rest
        else:
            (out_hbm, sidx, didx, rows, acc, gsem) = rest
        c = lax.axis_index("c")
        s = lax.axis_index("s")
        r0 = s * RPT
        pltpu.sync_copy(src_hbm.at[s], sidx)
        pltpu.sync_copy(dst_hbm.at[s], didx)
        pltpu.sync_copy(zeros_hbm.at[pl.ds(r0, RPT)], acc.at[pl.ds(r0, RPT)])
        if with_count:
            @pl.when(s == 0)
            def _():
                pltpu.sync_copy(zcnt_hbm, cnt.at[pl.ds(0, N_NODES)])
            for v in range(K // 16):
                ones[pl.ds(v * 16, 16)] = jnp.ones((16,), jnp.float32)
        plsc.subcore_barrier()

        def gfire(i, b):
            pltpu.async_copy(tbl_hbm.at[c].at[sidx.at[i]], rows[b], gsem[b])

        def gwait(i, b):
            pltpu.make_async_copy(tbl_hbm.at[c].at[sidx.at[i]], rows[b], gsem[b]).wait()

        def scatter(i, b):
            pltpu.sync_copy(rows[b], acc.at[didx.at[i]], add=True)
            if with_count:
                pltpu.sync_copy(ones, cnt.at[didx.at[i]], add=True)

        for b in range(4):
            gfire(b, b)

        def body(j, carry):
            a = 4 * j
            for b in range(4):
                gwait(a + b, b)
                scatter(a + b, b)
                gfire(a + 4 + b, b)
            return carry

        lax.fori_loop(0, (NCHUNK - 1) // 4 - 1, body, 0)
        a = NCHUNK - 5
        gwait(a, 0)
        scatter(a, 0)
        gfire(NCHUNK - 1, 0)
        for b in range(1, 4):
            gwait(a + b, b)
            scatter(a + b, b)
        gwait(NCHUNK - 1, 0)
        scatter(NCHUNK - 1, 0)

        plsc.subcore_barrier()
        pltpu.sync_copy(acc.at[pl.ds(r0, RPT)],
                        out_hbm.at[pl.ds(r0, RPT), pl.ds(c * CW, CW)])
        if with_count:
            @pl.when((c == 0) & (s == 0))
            def _():
                pltpu.sync_copy(cnt.at[pl.ds(0, N_NODES)], cnt_hbm)

    return agg


_agg_l1 = _make_sc_agg(True)
_agg_l2 = _make_sc_agg(False)

BLK = 1000

_WSPECS = [
    pl.BlockSpec((D_IN, D_IN), lambda i: (0, 0)),
    pl.BlockSpec((1, D_IN), lambda i: (0, 0)),
]


def _xr1_body(x_ref, wr, b, o_ref):
    o_ref[...] = (jnp.dot(x_ref[...], wr[...], preferred_element_type=jnp.float32)
                  + b[...])


def _xr1(x, wrT, b2d):
    return pl.pallas_call(
        _xr1_body,
        grid=(N_NODES // BLK,),
        in_specs=[pl.BlockSpec((BLK, D_IN), lambda i: (i, 0))] + _WSPECS,
        out_specs=pl.BlockSpec((BLK, D_IN), lambda i: (i, 0)),
        out_shape=jax.ShapeDtypeStruct((N_NODES, D_IN), jnp.float32),
    )(x, wrT, b2d)


def _xr2_body(h_ref, wr, b, o_ref):
    h = jnp.concatenate([h_ref[0], h_ref[1]], axis=1)
    o_ref[...] = (jnp.dot(h, wr[...], preferred_element_type=jnp.float32)
                  + b[...])


def _xr2(h_tbl, wrT, b2d):
    return pl.pallas_call(
        _xr2_body,
        grid=(N_NODES // BLK,),
        in_specs=[pl.BlockSpec((2, BLK, CW), lambda i: (0, i, 0))] + _WSPECS,
        out_specs=pl.BlockSpec((BLK, D_IN), lambda i: (i, 0)),
        out_shape=jax.ShapeDtypeStruct((N_NODES, D_IN), jnp.float32),
    )(h_tbl, wrT, b2d)


def _dense1_body(a_ref, cnt_ref, xr_ref, wl, h_ref, inv_ref):
    inv = 1.0 / jnp.maximum(cnt_ref[...], 1.0)
    mean = a_ref[...] * inv
    h = (jnp.dot(mean, wl[...], preferred_element_type=jnp.float32)
         + xr_ref[...])
    h = jnp.maximum(h, 0.0)
    h_ref[0] = h[:, :CW]
    h_ref[1] = h[:, CW:]
    inv_ref[...] = inv


def _dense1(acc, cnt2d, xr, wlT):
    return pl.pallas_call(
        _dense1_body,
        grid=(N_NODES // BLK,),
        in_specs=[
            pl.BlockSpec((BLK, D_IN), lambda i: (i, 0)),
            pl.BlockSpec((BLK, 1), lambda i: (i, 0)),
            pl.BlockSpec((BLK, D_IN), lambda i: (i, 0)),
            pl.BlockSpec((D_IN, D_IN), lambda i: (0, 0)),
        ],
        out_specs=[
            pl.BlockSpec((2, BLK, CW), lambda i: (0, i, 0)),
            pl.BlockSpec((BLK, 1), lambda i: (i, 0)),
        ],
        out_shape=[
            jax.ShapeDtypeStruct((2, N_NODES, CW), jnp.float32),
            jax.ShapeDtypeStruct((N_NODES, 1), jnp.float32),
        ],
    )(acc, cnt2d, xr, wlT)


def _dense2_body(a_ref, inv_ref, hr_ref, wl, o_ref):
    mean = a_ref[...] * inv_ref[...]
    o_ref[...] = (jnp.dot(mean, wl[...], preferred_element_type=jnp.float32)
                  + hr_ref[...])


def _dense2(acc, inv, hr, wlT):
    return pl.pallas_call(
        _dense2_body,
        grid=(N_NODES // BLK,),
        in_specs=[
            pl.BlockSpec((BLK, D_IN), lambda i: (i, 0)),
            pl.BlockSpec((BLK, 1), lambda i: (i, 0)),
            pl.BlockSpec((BLK, D_IN), lambda i: (i, 0)),
            pl.BlockSpec((D_IN, D_IN), lambda i: (0, 0)),
        ],
        out_specs=pl.BlockSpec((BLK, D_IN), lambda i: (i, 0)),
        out_shape=jax.ShapeDtypeStruct((N_NODES, D_IN), jnp.float32),
    )(acc, inv, hr, wlT)


def _pad_idx(v, trash):
    v = v.reshape(NS, EPW)
    npad = EPW_PAD - EPW
    if trash:
        fill = N_NODES + (jnp.arange(npad, dtype=jnp.int32) % 16)
    else:
        fill = jnp.zeros((npad,), jnp.int32)
    pad = jnp.broadcast_to(fill, (NS, npad))
    return jnp.concatenate([v, pad], axis=1).reshape(NS, NCHUNK, K)


def kernel(x, edge_index, Wl1, bl1, Wr1, Wl2, bl2, Wr2):
    src = _pad_idx(edge_index[0].astype(jnp.int32), False)
    dst = _pad_idx(edge_index[1].astype(jnp.int32), True)
    tbl1 = jnp.stack([x[:, :CW], x[:, CW:]])
    z64 = jnp.zeros((N_NODES, CW), jnp.float32)
    zcnt = jnp.zeros((N_NODES,), jnp.float32)

    acc1, cnt = _agg_l1(tbl1, src, dst, z64, zcnt)
    xr1 = _xr1(x, Wr1.T, bl1[None, :])
    h_tbl, inv = _dense1(acc1, cnt[:, None], xr1, Wl1.T)
    acc2, = _agg_l2(h_tbl, src, dst, z64)
    hr2 = _xr2(h_tbl, Wr2.T, bl2[None, :])
    out = _dense2(acc2, inv, hr2, Wl2.T)
    return out


# speedup vs baseline: 1.2892x; 1.0181x over previous
import functools

import jax
import jax.numpy as jnp
from jax import lax
from jax.experimental import pallas as pl
from jax.experimental.pallas import tpu as pltpu
from jax.experimental.pallas import tpu_sc as plsc

N_NODES = 10000
N_EDGES = 320000
D_IN = 128

NC = 2
NS = 16
CW = 64
K = 128
NB = 6
EPW = N_EDGES // NS
NCHUNK = -(-EPW // K)
EPW_PAD = NCHUNK * K
N_PAD = N_NODES + 16
RPT = N_NODES // NS


def _make_sc_agg(with_count):
    mesh = plsc.VectorSubcoreMesh(
        core_axis_name="c", subcore_axis_name="s", num_cores=NC, num_subcores=NS
    )
    out_type = [jax.ShapeDtypeStruct((N_NODES, NC * CW), jnp.float32)]
    scratch = [
        pltpu.VMEM((NCHUNK, K), jnp.int32),
        pltpu.VMEM((NCHUNK, K), jnp.int32),
        [pltpu.VMEM((K, CW), jnp.float32) for _ in range(NB)],
        pltpu.VMEM_SHARED((N_PAD, CW), jnp.float32),
        [pltpu.SemaphoreType.DMA for _ in range(NB)],
    ]
    if with_count:
        out_type.append(jax.ShapeDtypeStruct((N_NODES,), jnp.float32))
        scratch.append(pltpu.VMEM((K,), jnp.float32))
        scratch.append(pltpu.VMEM_SHARED((N_PAD,), jnp.float32))

    @functools.partial(
        pl.kernel,
        out_type=out_type,
        mesh=mesh,
        compiler_params=pltpu.CompilerParams(use_tc_tiling_on_sc=False),
        scratch_types=scratch,
    )
    def agg(tbl_hbm, src_hbm, dst_hbm, zeros_hbm, *rest):
        if with_count:
            (zcnt_hbm, out_hbm, cnt_hbm,
             sidx, didx, rows, acc, gsem, ones, cnt) = rest
        else:
            (out_hbm, sidx, didx, rows, acc, gsem) = rest
        c = lax.axis_index("c")
        s = lax.axis_index("s")
        r0 = s * RPT
        pltpu.sync_copy(src_hbm.at[s], sidx)
        pltpu.sync_copy(dst_hbm.at[s], didx)
        pltpu.sync_copy(zeros_hbm.at[pl.ds(r0, RPT)], acc.at[pl.ds(r0, RPT)])
        if with_count:
            @pl.when(s == 0)
            def _():
                pltpu.sync_copy(zcnt_hbm, cnt.at[pl.ds(0, N_NODES)])
            for v in range(K // 16):
                ones[pl.ds(v * 16, 16)] = jnp.ones((16,), jnp.float32)
        plsc.subcore_barrier()

        def gfire(i, b):
            pltpu.async_copy(tbl_hbm.at[c].at[sidx.at[i]], rows[b], gsem[b])

        def gwait(i, b):
            pltpu.make_async_copy(tbl_hbm.at[c].at[sidx.at[i]], rows[b], gsem[b]).wait()

        def scatter(i, b):
            pltpu.sync_copy(rows[b], acc.at[didx.at[i]], add=True)
            if with_count:
                pltpu.sync_copy(ones, cnt.at[didx.at[i]], add=True)

        for b in range(NB):
            gfire(b, b)

        def body(j, carry):
            a = NB * j
            for b in range(NB):
                gwait(a + b, b)
                scatter(a + b, b)
                gfire(a + NB + b, b)
            return carry

        lax.fori_loop(0, (NCHUNK - 1) // NB - 1, body, 0)
        a = NCHUNK - NB - 1
        gwait(a, 0)
        scatter(a, 0)
        gfire(NCHUNK - 1, 0)
        for b in range(1, NB):
            gwait(a + b, b)
            scatter(a + b, b)
        gwait(NCHUNK - 1, 0)
        scatter(NCHUNK - 1, 0)

        plsc.subcore_barrier()
        pltpu.sync_copy(acc.at[pl.ds(r0, RPT)],
                        out_hbm.at[pl.ds(r0, RPT), pl.ds(c * CW, CW)])
        if with_count:
            @pl.when((c == 0) & (s == 0))
            def _():
                pltpu.sync_copy(cnt.at[pl.ds(0, N_NODES)], cnt_hbm)

    return agg


_agg_l1 = _make_sc_agg(True)
_agg_l2 = _make_sc_agg(False)

BLK = 2000

_WSPECS = [
    pl.BlockSpec((D_IN, D_IN), lambda i: (0, 0)),
    pl.BlockSpec((1, D_IN), lambda i: (0, 0)),
    pl.BlockSpec((D_IN, D_IN), lambda i: (0, 0)),
]


def _dense1_body(a_ref, cnt_ref, x_ref, wl, b, wr, h_ref, inv_ref):
    inv = 1.0 / jnp.maximum(cnt_ref[...], 1.0)
    mean = a_ref[...] * inv
    h = (jnp.dot(mean, wl[...], preferred_element_type=jnp.float32)
         + b[...]
         + jnp.dot(x_ref[...], wr[...], preferred_element_type=jnp.float32))
    h = jnp.maximum(h, 0.0)
    h_ref[0] = h[:, :CW]
    h_ref[1] = h[:, CW:]
    inv_ref[...] = inv


def _dense1(acc, cnt2d, x, wlT, b2d, wrT):
    return pl.pallas_call(
        _dense1_body,
        grid=(N_NODES // BLK,),
        in_specs=[
            pl.BlockSpec((BLK, D_IN), lambda i: (i, 0)),
            pl.BlockSpec((BLK, 1), lambda i: (i, 0)),
            pl.BlockSpec((BLK, D_IN), lambda i: (i, 0)),
        ] + _WSPECS,
        out_specs=[
            pl.BlockSpec((2, BLK, CW), lambda i: (0, i, 0)),
            pl.BlockSpec((BLK, 1), lambda i: (i, 0)),
        ],
        out_shape=[
            jax.ShapeDtypeStruct((2, N_NODES, CW), jnp.float32),
            jax.ShapeDtypeStruct((N_NODES, 1), jnp.float32),
        ],
    )(acc, cnt2d, x, wlT, b2d, wrT)


def _dense2_body(a_ref, inv_ref, h_ref, wl, b, wr, o_ref):
    mean = a_ref[...] * inv_ref[...]
    h = jnp.concatenate([h_ref[0], h_ref[1]], axis=1)
    o_ref[...] = (jnp.dot(mean, wl[...], preferred_element_type=jnp.float32)
                  + b[...]
                  + jnp.dot(h, wr[...], preferred_element_type=jnp.float32))


def _dense2(acc, inv, h_tbl, wlT, b2d, wrT):
    return pl.pallas_call(
        _dense2_body,
        grid=(N_NODES // BLK,),
        in_specs=[
            pl.BlockSpec((BLK, D_IN), lambda i: (i, 0)),
            pl.BlockSpec((BLK, 1), lambda i: (i, 0)),
            pl.BlockSpec((2, BLK, CW), lambda i: (0, i, 0)),
        ] + _WSPECS,
        out_specs=pl.BlockSpec((BLK, D_IN), lambda i: (i, 0)),
        out_shape=jax.ShapeDtypeStruct((N_NODES, D_IN), jnp.float32),
    )(acc, inv, h_tbl, wlT, b2d, wrT)


def _pad_idx(v, trash):
    v = v.reshape(NS, EPW)
    npad = EPW_PAD - EPW
    if trash:
        fill = N_NODES + (jnp.arange(npad, dtype=jnp.int32) % 16)
    else:
        fill = jnp.zeros((npad,), jnp.int32)
    pad = jnp.broadcast_to(fill, (NS, npad))
    return jnp.concatenate([v, pad], axis=1).reshape(NS, NCHUNK, K)


def kernel(x, edge_index, Wl1, bl1, Wr1, Wl2, bl2, Wr2):
    src = _pad_idx(edge_index[0].astype(jnp.int32), False)
    dst = _pad_idx(edge_index[1].astype(jnp.int32), True)
    tbl1 = jnp.stack([x[:, :CW], x[:, CW:]])
    z64 = jnp.zeros((N_NODES, CW), jnp.float32)
    zcnt = jnp.zeros((N_NODES,), jnp.float32)

    acc1, cnt = _agg_l1(tbl1, src, dst, z64, zcnt)
    h_tbl, inv = _dense1(acc1, cnt[:, None], x, Wl1.T, bl1[None, :], Wr1.T)
    acc2, = _agg_l2(h_tbl, src, dst, z64)
    out = _dense2(acc2, inv, h_tbl, Wl2.T, bl2[None, :], Wr2.T)
    return out


# speedup vs baseline: 1.3038x; 1.0114x over previous
import functools

import jax
import jax.numpy as jnp
from jax import lax
from jax.experimental import pallas as pl
from jax.experimental.pallas import tpu as pltpu
from jax.experimental.pallas import tpu_sc as plsc

N_NODES = 10000
N_EDGES = 320000
D_IN = 128

NC = 2
NS = 16
CW = 64
K = 128
NB = 6
EPW = N_EDGES // NS
NCHUNK = -(-EPW // K)
EPW_PAD = NCHUNK * K
N_PAD = N_NODES + 16
RPT = N_NODES // NS


def _make_sc_agg(with_count):
    mesh = plsc.VectorSubcoreMesh(
        core_axis_name="c", subcore_axis_name="s", num_cores=NC, num_subcores=NS
    )
    out_type = [jax.ShapeDtypeStruct((N_NODES, NC * CW), jnp.float32)]
    scratch = [
        pltpu.VMEM((NCHUNK, K), jnp.int32),
        pltpu.VMEM((NCHUNK, K), jnp.int32),
        [pltpu.VMEM((K, CW), jnp.float32) for _ in range(NB)],
        pltpu.VMEM_SHARED((N_PAD, CW), jnp.float32),
        [pltpu.SemaphoreType.DMA for _ in range(NB)],
    ]
    if with_count:
        out_type.append(jax.ShapeDtypeStruct((N_NODES,), jnp.float32))
        scratch.append(pltpu.VMEM((K,), jnp.float32))
        scratch.append(pltpu.VMEM_SHARED((N_PAD,), jnp.float32))

    @functools.partial(
        pl.kernel,
        out_type=out_type,
        mesh=mesh,
        compiler_params=pltpu.CompilerParams(use_tc_tiling_on_sc=False),
        scratch_types=scratch,
    )
    def agg(tbl_hbm, src_hbm, dst_hbm, *rest):
        if with_count:
            (zcnt_hbm, out_hbm, cnt_hbm,
             sidx, didx, rows, acc, gsem, ones, cnt) = rest
        else:
            (out_hbm, sidx, didx, rows, acc, gsem) = rest
        c = lax.axis_index("c")
        s = lax.axis_index("s")
        r0 = s * RPT
        pltpu.sync_copy(src_hbm.at[s], sidx)
        pltpu.sync_copy(dst_hbm.at[s], didx)

        def zbody(r, carry):
            for u in range(CW // 16):
                rows[0][r, pl.ds(u * 16, 16)] = jnp.zeros((16,), jnp.float32)
            return carry

        lax.fori_loop(0, K, zbody, 0)
        for t in range(RPT // K):
            pltpu.sync_copy(rows[0], acc.at[pl.ds(r0 + t * K, K)])
        pltpu.sync_copy(rows[0].at[pl.ds(0, RPT % K)],
                        acc.at[pl.ds(r0 + (RPT // K) * K, RPT % K)])
        if with_count:
            @pl.when(s == 0)
            def _():
                pltpu.sync_copy(zcnt_hbm, cnt.at[pl.ds(0, N_NODES)])
            for v in range(K // 16):
                ones[pl.ds(v * 16, 16)] = jnp.ones((16,), jnp.float32)
        plsc.subcore_barrier()

        def gfire(i, b):
            pltpu.async_copy(tbl_hbm.at[c].at[sidx.at[i]], rows[b], gsem[b])

        def gwait(i, b):
            pltpu.make_async_copy(tbl_hbm.at[c].at[sidx.at[i]], rows[b], gsem[b]).wait()

        def scatter(i, b):
            pltpu.sync_copy(rows[b], acc.at[didx.at[i]], add=True)
            if with_count:
                pltpu.sync_copy(ones, cnt.at[didx.at[i]], add=True)

        for b in range(NB):
            gfire(b, b)

        def body(j, carry):
            a = NB * j
            for b in range(NB):
                gwait(a + b, b)
                scatter(a + b, b)
                gfire(a + NB + b, b)
            return carry

        lax.fori_loop(0, (NCHUNK - 1) // NB - 1, body, 0)
        a = NCHUNK - NB - 1
        gwait(a, 0)
        scatter(a, 0)
        gfire(NCHUNK - 1, 0)
        for b in range(1, NB):
            gwait(a + b, b)
            scatter(a + b, b)
        gwait(NCHUNK - 1, 0)
        scatter(NCHUNK - 1, 0)

        plsc.subcore_barrier()
        pltpu.sync_copy(acc.at[pl.ds(r0, RPT)],
                        out_hbm.at[pl.ds(r0, RPT), pl.ds(c * CW, CW)])
        if with_count:
            @pl.when((c == 0) & (s == 0))
            def _():
                pltpu.sync_copy(cnt.at[pl.ds(0, N_NODES)], cnt_hbm)

    return agg


_agg_l1 = _make_sc_agg(True)
_agg_l2 = _make_sc_agg(False)

BLK = 2000

_WSPECS = [
    pl.BlockSpec((D_IN, D_IN), lambda i: (0, 0)),
    pl.BlockSpec((1, D_IN), lambda i: (0, 0)),
    pl.BlockSpec((D_IN, D_IN), lambda i: (0, 0)),
]


def _dense1_body(a_ref, cnt_ref, x_ref, wl, b, wr, h_ref, inv_ref):
    inv = 1.0 / jnp.maximum(cnt_ref[...], 1.0)
    mean = a_ref[...] * inv
    h = (jnp.dot(mean, wl[...], preferred_element_type=jnp.float32)
         + b[...]
         + jnp.dot(x_ref[...], wr[...], preferred_element_type=jnp.float32))
    h = jnp.maximum(h, 0.0)
    h_ref[0] = h[:, :CW]
    h_ref[1] = h[:, CW:]
    inv_ref[...] = inv


def _dense1(acc, cnt2d, x, wlT, b2d, wrT):
    return pl.pallas_call(
        _dense1_body,
        grid=(N_NODES // BLK,),
        in_specs=[
            pl.BlockSpec((BLK, D_IN), lambda i: (i, 0)),
            pl.BlockSpec((BLK, 1), lambda i: (i, 0)),
            pl.BlockSpec((BLK, D_IN), lambda i: (i, 0)),
        ] + _WSPECS,
        out_specs=[
            pl.BlockSpec((2, BLK, CW), lambda i: (0, i, 0)),
            pl.BlockSpec((BLK, 1), lambda i: (i, 0)),
        ],
        out_shape=[
            jax.ShapeDtypeStruct((2, N_NODES, CW), jnp.float32),
            jax.ShapeDtypeStruct((N_NODES, 1), jnp.float32),
        ],
    )(acc, cnt2d, x, wlT, b2d, wrT)


def _dense2_body(a_ref, inv_ref, h_ref, wl, b, wr, o_ref):
    mean = a_ref[...] * inv_ref[...]
    h = jnp.concatenate([h_ref[0], h_ref[1]], axis=1)
    o_ref[...] = (jnp.dot(mean, wl[...], preferred_element_type=jnp.float32)
                  + b[...]
                  + jnp.dot(h, wr[...], preferred_element_type=jnp.float32))


def _dense2(acc, inv, h_tbl, wlT, b2d, wrT):
    return pl.pallas_call(
        _dense2_body,
        grid=(N_NODES // BLK,),
        in_specs=[
            pl.BlockSpec((BLK, D_IN), lambda i: (i, 0)),
            pl.BlockSpec((BLK, 1), lambda i: (i, 0)),
            pl.BlockSpec((2, BLK, CW), lambda i: (0, i, 0)),
        ] + _WSPECS,
        out_specs=pl.BlockSpec((BLK, D_IN), lambda i: (i, 0)),
        out_shape=jax.ShapeDtypeStruct((N_NODES, D_IN), jnp.float32),
    )(acc, inv, h_tbl, wlT, b2d, wrT)


def _pad_idx(v, trash):
    v = v.reshape(NS, EPW)
    npad = EPW_PAD - EPW
    if trash:
        fill = N_NODES + (jnp.arange(npad, dtype=jnp.int32) % 16)
    else:
        fill = jnp.zeros((npad,), jnp.int32)
    pad = jnp.broadcast_to(fill, (NS, npad))
    return jnp.concatenate([v, pad], axis=1).reshape(NS, NCHUNK, K)


def kernel(x, edge_index, Wl1, bl1, Wr1, Wl2, bl2, Wr2):
    src = _pad_idx(edge_index[0].astype(jnp.int32), False)
    dst = _pad_idx(edge_index[1].astype(jnp.int32), True)
    tbl1 = jnp.stack([x[:, :CW], x[:, CW:]])
    zcnt = jnp.zeros((N_NODES,), jnp.float32)

    acc1, cnt = _agg_l1(tbl1, src, dst, zcnt)
    h_tbl, inv = _dense1(acc1, cnt[:, None], x, Wl1.T, bl1[None, :], Wr1.T)
    acc2, = _agg_l2(h_tbl, src, dst)
    out = _dense2(acc2, inv, h_tbl, Wl2.T, bl2[None, :], Wr2.T)
    return out


# speedup vs baseline: 1.3299x; 1.0200x over previous
import functools

import jax
import jax.numpy as jnp
from jax import lax
from jax.experimental import pallas as pl
from jax.experimental.pallas import tpu as pltpu
from jax.experimental.pallas import tpu_sc as plsc

N_NODES = 10000
N_EDGES = 320000
D_IN = 128

NC = 2
NS = 16
CW = 64
K = 128
NB = 6
EPW = N_EDGES // NS
NCHUNK = -(-EPW // K)
EPW_PAD = NCHUNK * K
N_PAD = N_NODES + 16
RPT = N_NODES // NS


def _make_sc_agg(with_count):
    mesh = plsc.VectorSubcoreMesh(
        core_axis_name="c", subcore_axis_name="s", num_cores=NC, num_subcores=NS
    )
    out_type = [jax.ShapeDtypeStruct((N_NODES, NC * CW), jnp.float32)]
    scratch = [
        pltpu.VMEM((NCHUNK, K), jnp.int32),
        pltpu.VMEM((NCHUNK, K), jnp.int32),
        [pltpu.VMEM((K, CW), jnp.float32) for _ in range(NB)],
        pltpu.VMEM_SHARED((N_PAD, CW), jnp.float32),
        [pltpu.SemaphoreType.DMA for _ in range(NB)],
    ]
    if with_count:
        out_type.append(jax.ShapeDtypeStruct((N_NODES,), jnp.float32))
        scratch.append(pltpu.VMEM((K,), jnp.float32))
        scratch.append(pltpu.VMEM_SHARED((N_PAD,), jnp.float32))

    @functools.partial(
        pl.kernel,
        out_type=out_type,
        mesh=mesh,
        compiler_params=pltpu.CompilerParams(use_tc_tiling_on_sc=False),
        scratch_types=scratch,
    )
    def agg(tbl_hbm, src_hbm, dst_hbm, *rest):
        if with_count:
            (zcnt_hbm, out_hbm, cnt_hbm,
             sidx, didx, rows, acc, gsem, ones, cnt) = rest
        else:
            (out_hbm, sidx, didx, rows, acc, gsem) = rest
        c = lax.axis_index("c")
        s = lax.axis_index("s")
        r0 = s * RPT
        pltpu.async_copy(src_hbm.at[s], sidx, gsem[0])
        pltpu.async_copy(dst_hbm.at[s], didx, gsem[1])

        def zbody(r, carry):
            for u in range(CW // 16):
                rows[0][r, pl.ds(u * 16, 16)] = jnp.zeros((16,), jnp.float32)
            return carry

        lax.fori_loop(0, K, zbody, 0)
        nz = RPT // K
        for t in range(nz):
            pltpu.async_copy(rows[0], acc.at[pl.ds(r0 + t * K, K)], gsem[2])
        pltpu.async_copy(rows[0].at[pl.ds(0, RPT % K)],
                         acc.at[pl.ds(r0 + nz * K, RPT % K)], gsem[3])
        pltpu.make_async_copy(src_hbm.at[s], sidx, gsem[0]).wait()
        pltpu.make_async_copy(dst_hbm.at[s], didx, gsem[1]).wait()
        for t in range(nz):
            pltpu.make_async_copy(rows[0], acc.at[pl.ds(r0 + t * K, K)],
                                  gsem[2]).wait()
        pltpu.make_async_copy(rows[0].at[pl.ds(0, RPT % K)],
                              acc.at[pl.ds(r0 + nz * K, RPT % K)],
                              gsem[3]).wait()
        if with_count:
            @pl.when(s == 0)
            def _():
                pltpu.sync_copy(zcnt_hbm, cnt.at[pl.ds(0, N_NODES)])
            for v in range(K // 16):
                ones[pl.ds(v * 16, 16)] = jnp.ones((16,), jnp.float32)
        plsc.subcore_barrier()

        def gfire(i, b):
            pltpu.async_copy(tbl_hbm.at[c].at[sidx.at[i]], rows[b], gsem[b])

        def gwait(i, b):
            pltpu.make_async_copy(tbl_hbm.at[c].at[sidx.at[i]], rows[b], gsem[b]).wait()

        def scatter(i, b):
            pltpu.sync_copy(rows[b], acc.at[didx.at[i]], add=True)
            if with_count:
                pltpu.sync_copy(ones, cnt.at[didx.at[i]], add=True)

        for b in range(NB):
            gfire(b, b)

        def body(j, carry):
            a = NB * j
            for b in range(NB):
                gwait(a + b, b)
                scatter(a + b, b)
                gfire(a + NB + b, b)
            return carry

        lax.fori_loop(0, (NCHUNK - 1) // NB - 1, body, 0)
        a = NCHUNK - NB - 1
        gwait(a, 0)
        scatter(a, 0)
        gfire(NCHUNK - 1, 0)
        for b in range(1, NB):
            gwait(a + b, b)
            scatter(a + b, b)
        gwait(NCHUNK - 1, 0)
        scatter(NCHUNK - 1, 0)

        plsc.subcore_barrier()
        pltpu.sync_copy(acc.at[pl.ds(r0, RPT)],
                        out_hbm.at[pl.ds(r0, RPT), pl.ds(c * CW, CW)])
        if with_count:
            @pl.when((c == 0) & (s == 0))
            def _():
                pltpu.sync_copy(cnt.at[pl.ds(0, N_NODES)], cnt_hbm)

    return agg


_agg_l1 = _make_sc_agg(True)
_agg_l2 = _make_sc_agg(False)

BLK = 2000

_WSPECS = [
    pl.BlockSpec((D_IN, D_IN), lambda i: (0, 0)),
    pl.BlockSpec((1, D_IN), lambda i: (0, 0)),
    pl.BlockSpec((D_IN, D_IN), lambda i: (0, 0)),
]


def _dense1_body(a_ref, cnt_ref, x_ref, wl, b, wr, h_ref, inv_ref):
    inv = 1.0 / jnp.maximum(cnt_ref[...], 1.0)
    mean = a_ref[...] * inv
    h = (jnp.dot(mean, wl[...], preferred_element_type=jnp.float32)
         + b[...]
         + jnp.dot(x_ref[...], wr[...], preferred_element_type=jnp.float32))
    h = jnp.maximum(h, 0.0)
    h_ref[0] = h[:, :CW]
    h_ref[1] = h[:, CW:]
    inv_ref[...] = inv


def _dense1(acc, cnt2d, x, wlT, b2d, wrT):
    return pl.pallas_call(
        _dense1_body,
        grid=(N_NODES // BLK,),
        in_specs=[
            pl.BlockSpec((BLK, D_IN), lambda i: (i, 0)),
            pl.BlockSpec((BLK, 1), lambda i: (i, 0)),
            pl.BlockSpec((BLK, D_IN), lambda i: (i, 0)),
        ] + _WSPECS,
        out_specs=[
            pl.BlockSpec((2, BLK, CW), lambda i: (0, i, 0)),
            pl.BlockSpec((BLK, 1), lambda i: (i, 0)),
        ],
        out_shape=[
            jax.ShapeDtypeStruct((2, N_NODES, CW), jnp.float32),
            jax.ShapeDtypeStruct((N_NODES, 1), jnp.float32),
        ],
    )(acc, cnt2d, x, wlT, b2d, wrT)


def _dense2_body(a_ref, inv_ref, h_ref, wl, b, wr, o_ref):
    mean = a_ref[...] * inv_ref[...]
    h = jnp.concatenate([h_ref[0], h_ref[1]], axis=1)
    o_ref[...] = (jnp.dot(mean, wl[...], preferred_element_type=jnp.float32)
                  + b[...]
                  + jnp.dot(h, wr[...], preferred_element_type=jnp.float32))


def _dense2(acc, inv, h_tbl, wlT, b2d, wrT):
    return pl.pallas_call(
        _dense2_body,
        grid=(N_NODES // BLK,),
        in_specs=[
            pl.BlockSpec((BLK, D_IN), lambda i: (i, 0)),
            pl.BlockSpec((BLK, 1), lambda i: (i, 0)),
            pl.BlockSpec((2, BLK, CW), lambda i: (0, i, 0)),
        ] + _WSPECS,
        out_specs=pl.BlockSpec((BLK, D_IN), lambda i: (i, 0)),
        out_shape=jax.ShapeDtypeStruct((N_NODES, D_IN), jnp.float32),
    )(acc, inv, h_tbl, wlT, b2d, wrT)


def _pad_idx(v, trash):
    v = v.reshape(NS, EPW)
    npad = EPW_PAD - EPW
    if trash:
        fill = N_NODES + (jnp.arange(npad, dtype=jnp.int32) % 16)
    else:
        fill = jnp.zeros((npad,), jnp.int32)
    pad = jnp.broadcast_to(fill, (NS, npad))
    return jnp.concatenate([v, pad], axis=1).reshape(NS, NCHUNK, K)


def kernel(x, edge_index, Wl1, bl1, Wr1, Wl2, bl2, Wr2):
    src = _pad_idx(edge_index[0].astype(jnp.int32), False)
    dst = _pad_idx(edge_index[1].astype(jnp.int32), True)
    tbl1 = jnp.stack([x[:, :CW], x[:, CW:]])
    zcnt = jnp.zeros((N_NODES,), jnp.float32)

    acc1, cnt = _agg_l1(tbl1, src, dst, zcnt)
    h_tbl, inv = _dense1(acc1, cnt[:, None], x, Wl1.T, bl1[None, :], Wr1.T)
    acc2, = _agg_l2(h_tbl, src, dst)
    out = _dense2(acc2, inv, h_tbl, Wl2.T, bl2[None, :], Wr2.T)
    return out
